# Initial kernel scaffold; baseline (speedup 1.0000x reference)
import jax
import jax.numpy as jnp
from jax.experimental import pallas as pl


def kernel(x, codebook):
    raise NotImplementedError("write your pallas kernel here")


# speedup vs baseline: 3.5108x; 3.5108x over previous
import functools

import jax
import jax.numpy as jnp
from jax.experimental import pallas as pl
from jax.experimental.pallas import tpu as pltpu

CB = 8192
D = 256
BETA = 0.25
BLK = 256


def _vq_step(xf_ref, x2_ref, cb_ref, c2_ref,
             enc_ref, idx_ref, xq_ref, counts_ref, loss_ref, perp_ref,
             acc_ref, *, nblk, n_rows):
    i = pl.program_id(0)

    @pl.when(i == 0)
    def _init():
        counts_ref[...] = jnp.zeros_like(counts_ref)
        acc_ref[...] = jnp.zeros_like(acc_ref)

    x = xf_ref[...]
    cb = cb_ref[...]
    x2 = x2_ref[:, 0:1]
    c2 = c2_ref[...]

    xc = jax.lax.dot_general(x, cb, (((1,), (1,)), ((), ())),
                             preferred_element_type=jnp.float32)
    d = (x2 + c2) - 2.0 * xc

    dmin = jnp.min(d, axis=1, keepdims=True)
    col = jax.lax.broadcasted_iota(jnp.int32, (BLK, CB), 1)
    idx = jnp.min(jnp.where(d == dmin, col, CB), axis=1)
    idx_ref[...] = idx.reshape(1, 1, BLK)

    onehot = (col == idx[:, None]).astype(jnp.float32)
    enc_ref[...] = onehot
    counts_ref[...] += jnp.sum(onehot, axis=0, keepdims=True)

    xq = jax.lax.dot_general(onehot, cb, (((1,), (0,)), ((), ())),
                             preferred_element_type=jnp.float32,
                             precision=jax.lax.Precision.HIGHEST)
    xq_ref[...] = x + (xq - x)

    r = x - xq
    acc_ref[0:1, :] += jnp.sum(r, axis=0, keepdims=True)
    acc_ref[1:2, :] += jnp.sum(r * r, axis=0, keepdims=True)

    @pl.when(i == nblk - 1)
    def _finish():
        total = jnp.float32(n_rows * D)
        s1 = jnp.sum(acc_ref[0:1, :])
        s2 = jnp.sum(acc_ref[1:2, :])
        loss_ref[...] = (BETA * (s1 / total) + s2 / total).reshape(1, 1)
        e_mean = counts_ref[...] / jnp.float32(n_rows)
        ent = -jnp.sum(e_mean * jnp.log(e_mean + 1e-10))
        perp_ref[...] = jnp.exp(ent).reshape(1, 1)


@jax.jit
def kernel(x, codebook):
    b, c, h, w = x.shape
    xp = jnp.transpose(x, (0, 2, 3, 1))
    xf = xp.reshape(-1, c)
    n = xf.shape[0]
    nblk = n // BLK
    x2 = jnp.sum(xf ** 2, axis=1, keepdims=True)
    x2b = jnp.broadcast_to(x2, (n, 128))
    c2 = jnp.sum(codebook ** 2, axis=1).reshape(1, CB)

    grid = (nblk,)
    out_shapes = (
        jax.ShapeDtypeStruct((n, CB), jnp.float32),
        jax.ShapeDtypeStruct((nblk, 1, BLK), jnp.int32),
        jax.ShapeDtypeStruct((n, D), jnp.float32),
        jax.ShapeDtypeStruct((1, CB), jnp.float32),
        jax.ShapeDtypeStruct((1, 1), jnp.float32),
        jax.ShapeDtypeStruct((1, 1), jnp.float32),
    )
    enc, idx3, xqf, _counts, loss, perp = pl.pallas_call(
        functools.partial(_vq_step, nblk=nblk, n_rows=n),
        grid=grid,
        in_specs=[
            pl.BlockSpec((BLK, D), lambda i: (i, 0)),
            pl.BlockSpec((BLK, 128), lambda i: (i, 0)),
            pl.BlockSpec((CB, D), lambda i: (0, 0)),
            pl.BlockSpec((1, CB), lambda i: (0, 0)),
        ],
        out_specs=(
            pl.BlockSpec((BLK, CB), lambda i: (i, 0)),
            pl.BlockSpec((1, 1, BLK), lambda i: (i, 0, 0)),
            pl.BlockSpec((BLK, D), lambda i: (i, 0)),
            pl.BlockSpec((1, CB), lambda i: (0, 0)),
            pl.BlockSpec((1, 1), lambda i: (0, 0)),
            pl.BlockSpec((1, 1), lambda i: (0, 0)),
        ),
        out_shape=out_shapes,
        scratch_shapes=[pltpu.VMEM((2, D), jnp.float32)],
    )(xf, x2b, codebook, c2)

    min_encoding_indices = idx3.reshape(n, 1)
    x_quantized = jnp.transpose(xqf.reshape(b, h, w, c), (0, 3, 1, 2))
    return (x_quantized, loss.reshape(()), perp.reshape(()),
            enc, min_encoding_indices)


# speedup vs baseline: 7.0992x; 2.0221x over previous
import functools

import jax
from jax import lax
import jax.numpy as jnp
from jax.experimental import pallas as pl
from jax.experimental.pallas import tpu as pltpu
from jax.experimental.pallas import tpu_sc as plsc

CB = 8192
D = 256
BETA = 0.25
BLK = 256


def _vq_step(xf_ref, x2_ref, cb_ref, c2_ref, rs_ref,
             enc_ref, idx_ref, counts_ref, loss_ref, perp_ref,
             accx_ref, accd_ref, *, nblk, n_rows):
    i = pl.program_id(0)

    @pl.when(i == 0)
    def _init():
        counts_ref[...] = jnp.zeros_like(counts_ref)
        accx_ref[...] = jnp.zeros_like(accx_ref)
        accd_ref[...] = jnp.zeros_like(accd_ref)

    x = xf_ref[...]
    cb = cb_ref[...]
    x2 = x2_ref[:, 0:1]
    c2 = c2_ref[...]

    xc = jax.lax.dot_general(x, cb, (((1,), (1,)), ((), ())),
                             preferred_element_type=jnp.float32)
    d = (x2 + c2) - 2.0 * xc

    dmin = jnp.min(d, axis=1, keepdims=True)
    col = jax.lax.broadcasted_iota(jnp.int32, (BLK, CB), 1)
    idx = jnp.min(jnp.where(d == dmin, col, CB), axis=1)
    idx_ref[...] = idx.reshape(1, 1, BLK)

    onehot = (col == idx[:, None]).astype(jnp.float32)
    enc_ref[...] = onehot
    counts_ref[...] += jnp.sum(onehot, axis=0, keepdims=True)

    accx_ref[...] += jnp.sum(x, axis=0, keepdims=True)
    accd_ref[...] += jnp.sum(dmin).reshape(1, 1)

    @pl.when(i == nblk - 1)
    def _finish():
        total = jnp.float32(n_rows * D)
        counts = counts_ref[...]
        s1 = jnp.sum(accx_ref[...]) - jnp.sum(counts * rs_ref[...])
        s2 = accd_ref[0, 0]
        loss_ref[...] = (BETA * (s1 / total) + s2 / total).reshape(1, 1)
        e_mean = counts / jnp.float32(n_rows)
        ent = -jnp.sum(e_mean * jnp.log(e_mean + 1e-10))
        perp_ref[...] = jnp.exp(ent).reshape(1, 1)


@functools.cache
def _make_sc_gather(n_rows):
    info = plsc.get_sparse_core_info()
    nc, ns = info.num_cores, info.num_subcores
    nw = nc * ns
    rows_per_w = n_rows // nw

    @functools.partial(
        pl.kernel,
        mesh=plsc.VectorSubcoreMesh(core_axis_name="c", subcore_axis_name="s"),
        out_type=jax.ShapeDtypeStruct((n_rows, D), jnp.float32),
        scratch_types=[
            pltpu.VMEM((rows_per_w,), jnp.int32),
            pltpu.VMEM((rows_per_w, D), jnp.float32),
            pltpu.SemaphoreType.DMA,
        ],
    )
    def _sc_gather(table_hbm, idx_hbm, out_hbm, idx_v, rows_v, sem):
        wid = lax.axis_index("s") * nc + lax.axis_index("c")
        base = wid * rows_per_w
        pltpu.sync_copy(idx_hbm.at[pl.ds(base, rows_per_w)], idx_v)
        pltpu.async_copy(table_hbm.at[idx_v], rows_v, sem).wait()
        pltpu.sync_copy(rows_v, out_hbm.at[pl.ds(base, rows_per_w)])

    return _sc_gather


@jax.jit
def kernel(x, codebook):
    b, c, h, w = x.shape
    xp = jnp.transpose(x, (0, 2, 3, 1))
    xf = xp.reshape(-1, c)
    n = xf.shape[0]
    nblk = n // BLK
    x2 = jnp.sum(xf ** 2, axis=1, keepdims=True)
    x2b = jnp.broadcast_to(x2, (n, 128))
    c2 = jnp.sum(codebook ** 2, axis=1).reshape(1, CB)
    rs = jnp.sum(codebook, axis=1).reshape(1, CB)

    grid = (nblk,)
    out_shapes = (
        jax.ShapeDtypeStruct((n, CB), jnp.float32),
        jax.ShapeDtypeStruct((nblk, 1, BLK), jnp.int32),
        jax.ShapeDtypeStruct((1, CB), jnp.float32),
        jax.ShapeDtypeStruct((1, 1), jnp.float32),
        jax.ShapeDtypeStruct((1, 1), jnp.float32),
    )
    enc, idx3, _counts, loss, perp = pl.pallas_call(
        functools.partial(_vq_step, nblk=nblk, n_rows=n),
        grid=grid,
        in_specs=[
            pl.BlockSpec((BLK, D), lambda i: (i, 0)),
            pl.BlockSpec((BLK, 128), lambda i: (i, 0)),
            pl.BlockSpec((CB, D), lambda i: (0, 0)),
            pl.BlockSpec((1, CB), lambda i: (0, 0)),
            pl.BlockSpec((1, CB), lambda i: (0, 0)),
        ],
        out_specs=(
            pl.BlockSpec((BLK, CB), lambda i: (i, 0)),
            pl.BlockSpec((1, 1, BLK), lambda i: (i, 0, 0)),
            pl.BlockSpec((1, CB), lambda i: (0, 0)),
            pl.BlockSpec((1, 1), lambda i: (0, 0)),
            pl.BlockSpec((1, 1), lambda i: (0, 0)),
        ),
        out_shape=out_shapes,
        scratch_shapes=[pltpu.VMEM((1, D), jnp.float32),
                        pltpu.VMEM((1, 1), jnp.float32)],
    )(xf, x2b, codebook, c2, rs)

    min_encoding_indices = idx3.reshape(n, 1)
    xqf = _make_sc_gather(n)(codebook, idx3.reshape(n))
    x_quantized = jnp.transpose(xqf.reshape(b, h, w, c), (0, 3, 1, 2))
    return (x_quantized, loss.reshape(()), perp.reshape(()),
            enc, min_encoding_indices)


# speedup vs baseline: 8.0990x; 1.1408x over previous
import functools

import jax
from jax import lax
import jax.numpy as jnp
from jax.experimental import pallas as pl
from jax.experimental.pallas import tpu as pltpu
from jax.experimental.pallas import tpu_sc as plsc

CB = 8192
D = 256
BETA = 0.25
BLK = 512


def _vq_step(xf_ref, x2_ref, cb_ref, c2_ref, rs_ref,
             enc_ref, idx_ref, counts_ref, loss_ref, perp_ref,
             accx_ref, accd_ref, *, nblk, n_rows):
    i = pl.program_id(0)

    @pl.when(i == 0)
    def _init():
        counts_ref[...] = jnp.zeros_like(counts_ref)
        accx_ref[...] = jnp.zeros_like(accx_ref)
        accd_ref[...] = jnp.zeros_like(accd_ref)

    x = xf_ref[...]
    cb = cb_ref[...]
    x2 = x2_ref[:, 0:1]
    c2 = c2_ref[...]

    xc = jax.lax.dot_general(x, cb, (((1,), (1,)), ((), ())),
                             preferred_element_type=jnp.float32)
    d = (x2 + c2) - 2.0 * xc

    dmin = jnp.min(d, axis=1, keepdims=True)
    col = jax.lax.broadcasted_iota(jnp.int32, (BLK, CB), 1)
    idx = jnp.min(jnp.where(d == dmin, col, CB), axis=1)
    idx_ref[...] = idx.reshape(1, 1, BLK)

    onehot = (col == idx[:, None]).astype(jnp.float32)
    enc_ref[...] = onehot
    ones_row = jnp.ones((1, BLK), jnp.float32)
    counts_ref[...] += jax.lax.dot_general(
        ones_row, onehot, (((1,), (0,)), ((), ())),
        preferred_element_type=jnp.float32)

    accx_ref[...] += jnp.sum(x, axis=0, keepdims=True)
    accd_ref[...] += jnp.sum(dmin).reshape(1, 1)

    @pl.when(i == nblk - 1)
    def _finish():
        total = jnp.float32(n_rows * D)
        counts = counts_ref[...]
        s1 = jnp.sum(accx_ref[...]) - jnp.sum(counts * rs_ref[...])
        s2 = accd_ref[0, 0]
        loss_ref[...] = (BETA * (s1 / total) + s2 / total).reshape(1, 1)
        e_mean = counts / jnp.float32(n_rows)
        ent = -jnp.sum(e_mean * jnp.log(e_mean + 1e-10))
        perp_ref[...] = jnp.exp(ent).reshape(1, 1)


@functools.cache
def _make_sc_gather(n_rows):
    info = plsc.get_sparse_core_info()
    nc, ns = info.num_cores, info.num_subcores
    nw = nc * ns
    rows_per_w = n_rows // nw

    @functools.partial(
        pl.kernel,
        mesh=plsc.VectorSubcoreMesh(core_axis_name="c", subcore_axis_name="s"),
        out_type=jax.ShapeDtypeStruct((n_rows, D), jnp.float32),
        scratch_types=[
            pltpu.VMEM((rows_per_w,), jnp.int32),
            pltpu.VMEM((rows_per_w, D), jnp.float32),
            pltpu.SemaphoreType.DMA,
        ],
    )
    def _sc_gather(table_hbm, idx_hbm, out_hbm, idx_v, rows_v, sem):
        wid = lax.axis_index("s") * nc + lax.axis_index("c")
        base = wid * rows_per_w
        pltpu.sync_copy(idx_hbm.at[pl.ds(base, rows_per_w)], idx_v)
        pltpu.async_copy(table_hbm.at[idx_v], rows_v, sem).wait()
        pltpu.sync_copy(rows_v, out_hbm.at[pl.ds(base, rows_per_w)])

    return _sc_gather


@jax.jit
def kernel(x, codebook):
    b, c, h, w = x.shape
    xp = jnp.transpose(x, (0, 2, 3, 1))
    xf = xp.reshape(-1, c)
    n = xf.shape[0]
    nblk = n // BLK
    x2 = jnp.sum(xf ** 2, axis=1, keepdims=True)
    x2b = jnp.broadcast_to(x2, (n, 128))
    c2 = jnp.sum(codebook ** 2, axis=1).reshape(1, CB)
    rs = jnp.sum(codebook, axis=1).reshape(1, CB)

    grid = (nblk,)
    out_shapes = (
        jax.ShapeDtypeStruct((n, CB), jnp.float32),
        jax.ShapeDtypeStruct((nblk, 1, BLK), jnp.int32),
        jax.ShapeDtypeStruct((1, CB), jnp.float32),
        jax.ShapeDtypeStruct((1, 1), jnp.float32),
        jax.ShapeDtypeStruct((1, 1), jnp.float32),
    )
    enc, idx3, _counts, loss, perp = pl.pallas_call(
        functools.partial(_vq_step, nblk=nblk, n_rows=n),
        grid=grid,
        in_specs=[
            pl.BlockSpec((BLK, D), lambda i: (i, 0)),
            pl.BlockSpec((BLK, 128), lambda i: (i, 0)),
            pl.BlockSpec((CB, D), lambda i: (0, 0)),
            pl.BlockSpec((1, CB), lambda i: (0, 0)),
            pl.BlockSpec((1, CB), lambda i: (0, 0)),
        ],
        out_specs=(
            pl.BlockSpec((BLK, CB), lambda i: (i, 0)),
            pl.BlockSpec((1, 1, BLK), lambda i: (i, 0, 0)),
            pl.BlockSpec((1, CB), lambda i: (0, 0)),
            pl.BlockSpec((1, 1), lambda i: (0, 0)),
            pl.BlockSpec((1, 1), lambda i: (0, 0)),
        ),
        out_shape=out_shapes,
        scratch_shapes=[pltpu.VMEM((1, D), jnp.float32),
                        pltpu.VMEM((1, 1), jnp.float32)],
    )(xf, x2b, codebook, c2, rs)

    min_encoding_indices = idx3.reshape(n, 1)
    xqf = _make_sc_gather(n)(codebook, idx3.reshape(n))
    x_quantized = jnp.transpose(xqf.reshape(b, h, w, c), (0, 3, 1, 2))
    return (x_quantized, loss.reshape(()), perp.reshape(()),
            enc, min_encoding_indices)
